# Initial kernel scaffold; baseline (speedup 1.0000x reference)
import jax
import jax.numpy as jnp
from jax.experimental import pallas as pl


def kernel(P, B, row, col):
    raise NotImplementedError("write your pallas kernel here")


# speedup vs baseline: 10.4587x; 10.4587x over previous
import jax
import jax.numpy as jnp
from jax.experimental import pallas as pl
from jax.experimental.pallas import tpu as pltpu

_R = 100
_C = 1000
_S = 1000
_NE = 1600000
_EB = 6400


def _shift_right_flat(x):
    lastcol = x[:, _C - 1:_C]
    rolled = jnp.concatenate([lastcol[_R - 1:_R], lastcol[:_R - 1]], axis=0)
    return jnp.concatenate([rolled, x[:, :_C - 1]], axis=1)


def _shift_left_flat(x):
    firstcol = x[:, 0:1]
    rolled = jnp.concatenate([firstcol[1:], firstcol[0:1]], axis=0)
    return jnp.concatenate([x[:, 1:], rolled], axis=1)


def _node_kernel(bx_ref, px_ref, py_ref, pz_ref,
                 tn_ref, cd_ref, cm_ref, dn_ref, cx_ref,
                 opx_ref, opy_ref, opz_ref, odx_ref, ody_ref, odz_ref,
                 seg_ref,
                 fp_s, g1_s, g2_s, mx_s):
    f32 = jnp.float32
    Bv = bx_ref[...]
    Pxv = px_ref[...]
    Pyv = py_ref[...]
    Pzv = pz_ref[...]

    prevB = _shift_right_flat(Bv)
    nextB = _shift_left_flat(Bv)
    fp2 = (Bv != prevB).astype(f32)
    lp2 = (Bv != nextB).astype(f32)
    fp_s[...] = fp2

    keep = 1.0 - lp2
    dx0 = (_shift_left_flat(Pxv) - Pxv) * keep
    dy0 = (_shift_left_flat(Pyv) - Pyv) * keep
    dz0 = (_shift_left_flat(Pzv) - Pzv) * keep
    odx_ref[...] = dx0
    ody_ref[...] = dy0
    odz_ref[...] = dz0

    sub_i = jax.lax.broadcasted_iota(jnp.int32, (_S, _S), 0)
    lane_i = jax.lax.broadcasted_iota(jnp.int32, (_S, _S), 1)
    Tincl = (sub_i <= lane_i).astype(f32)
    Tstrict = (sub_i < lane_i).astype(f32)
    iota_row = jax.lax.broadcasted_iota(jnp.int32, (1, _S), 1)
    iota_col = jax.lax.broadcasted_iota(jnp.int32, (_S, 1), 0)

    B00 = Bv[0:1, 0:1]
    fp0f = (Bv[0:1, 0:1] != Bv[_R - 1:_R, _C - 1:_C]).astype(f32)

    def loop1(r, acc):
        brow = bx_ref[pl.ds(r, 1), :]
        oh_sn = (iota_col == brow).astype(f32)
        oh_ns = oh_sn.T
        pxr = px_ref[pl.ds(r, 1), :]
        pyr = py_ref[pl.ds(r, 1), :]
        pzr = pz_ref[pl.ds(r, 1), :]
        dxr = odx_ref[pl.ds(r, 1), :]
        dyr = ody_ref[pl.ds(r, 1), :]
        dzr = odz_ref[pl.ds(r, 1), :]
        ones = jnp.ones((1, _C), f32)
        M = jnp.concatenate(
            [ones, pxr, pyr, pzr, pxr * pxr, pyr * pyr, pzr * pzr,
             dxr, dyr, dzr, jnp.zeros((6, _C), f32)], axis=0)
        return acc + jnp.dot(M, oh_ns, preferred_element_type=f32, precision=jax.lax.Precision.HIGHEST)

    segA = jax.lax.fori_loop(0, _R, loop1, jnp.zeros((16, _S), f32))

    L = segA[0:1]
    cnt = jnp.maximum(L, 1.0)
    mP = segA[1:4] / cnt
    mP2 = segA[4:7] / cnt
    var = jnp.maximum(mP2 - mP * mP, 0.0)
    pstd = jnp.sqrt(jnp.sum(var, axis=0, keepdims=True))
    um = segA[7:10] / cnt
    un = um / jnp.sqrt(1e-05 + jnp.sum(um * um, axis=0, keepdims=True))
    start = jnp.dot(L, Tstrict, preferred_element_type=f32, precision=jax.lax.Precision.HIGHEST)
    logL = jnp.log(L)
    seg_ref[...] = jnp.concatenate(
        [pstd, logL, un, jnp.zeros((3, _S), f32)], axis=0)

    vals1 = jnp.concatenate([L, start, pstd], axis=0)

    def loop2(r, acc):
        brow = bx_ref[pl.ds(r, 1), :]
        oh_sn = (iota_col == brow).astype(f32)
        oh_ns = oh_sn.T
        bc = jnp.dot(vals1, oh_sn, preferred_element_type=f32, precision=jax.lax.Precision.HIGHEST)
        Lr = bc[0:1]
        startr = bc[1:2]
        sr = bc[2:3]
        pxr = px_ref[pl.ds(r, 1), :] / sr
        pyr = py_ref[pl.ds(r, 1), :] / sr
        pzr = pz_ref[pl.ds(r, 1), :] / sr
        opx_ref[pl.ds(r, 1), :] = pxr
        opy_ref[pl.ds(r, 1), :] = pyr
        opz_ref[pl.ds(r, 1), :] = pzr
        dxr = odx_ref[pl.ds(r, 1), :] / sr
        dyr = ody_ref[pl.ds(r, 1), :] / sr
        dzr = odz_ref[pl.ds(r, 1), :] / sr
        odx_ref[pl.ds(r, 1), :] = dxr
        ody_ref[pl.ds(r, 1), :] = dyr
        odz_ref[pl.ds(r, 1), :] = dzr
        dnr = jnp.sqrt(1e-05 + dxr * dxr + dyr * dyr + dzr * dzr)
        dn_ref[pl.ds(r, 1), :] = dnr
        fpr = fp_s[pl.ds(r, 1), :]
        M = jnp.concatenate(
            [dnr, dnr * dnr, dnr * fpr, jnp.zeros((5, _C), f32)], axis=0)
        lanef = jax.lax.broadcasted_iota(jnp.int32, (1, _C), 1).astype(f32)
        corrr = (1.0 - fp0f) * (brow == B00).astype(f32)
        tr = (r.astype(f32) * _C + lanef + 1.0) - startr - corrr
        tn_ref[pl.ds(r, 1), :] = tr / Lr
        return acc + jnp.dot(M, oh_ns, preferred_element_type=f32, precision=jax.lax.Precision.HIGHEST)

    segB = jax.lax.fori_loop(0, _R, loop2, jnp.zeros((8, _S), f32))

    sdn = segB[0:1]
    sdn2 = segB[1:2]
    fvm = segB[2:3]
    prev1 = jnp.dot(sdn, Tstrict, preferred_element_type=f32, precision=jax.lax.Precision.HIGHEST)
    prev2 = jnp.dot(sdn2, Tstrict, preferred_element_type=f32, precision=jax.lax.Precision.HIGHEST)
    vals2 = jnp.concatenate([prev1, prev2, fvm], axis=0)

    dnfull = dn_ref[...]
    r_i = jax.lax.broadcasted_iota(jnp.int32, (_R, _R), 0)
    r_j = jax.lax.broadcasted_iota(jnp.int32, (_R, _R), 1)
    Lstrict = (r_j < r_i).astype(f32)

    def cumsum2d(x):
        y = jnp.dot(x, Tincl, preferred_element_type=f32, precision=jax.lax.Precision.HIGHEST)
        offs = jnp.dot(Lstrict, y[:, _C - 1:_C], preferred_element_type=f32, precision=jax.lax.Precision.HIGHEST)
        return y + offs

    g1_s[...] = cumsum2d(dnfull)
    g2_s[...] = cumsum2d(dnfull * dnfull)

    y = dnfull
    for sft in (1, 2, 4, 8, 16, 32, 64, 128, 256, 512):
        y = jnp.maximum(
            y, jnp.concatenate(
                [jnp.zeros((_R, sft), f32), y[:, :_C - sft]], axis=1))
    z = y[:, _C - 1:_C]
    for sft in (1, 2, 4, 8, 16, 32, 64):
        z = jnp.maximum(
            z, jnp.concatenate(
                [jnp.zeros((sft, 1), f32), z[:_R - sft]], axis=0))
    excl = jnp.concatenate([jnp.zeros((1, 1), f32), z[:_R - 1]], axis=0)
    mx_s[...] = jnp.maximum(y, excl)

    dn00 = dnfull[0:1, 0:1]

    def loop3(r, acc):
        brow = bx_ref[pl.ds(r, 1), :]
        oh_sn = (iota_col == brow).astype(f32)
        bc = jnp.dot(vals2, oh_sn, preferred_element_type=f32, precision=jax.lax.Precision.HIGHEST)
        corrm = (1.0 - fp0f) * (brow == B00).astype(f32)
        g1r = g1_s[pl.ds(r, 1), :]
        g2r = g2_s[pl.ds(r, 1), :]
        mxr = mx_s[pl.ds(r, 1), :]
        cd_ref[pl.ds(r, 1), :] = g1r - bc[0:1] - corrm * dn00
        cm_ref[pl.ds(r, 1), :] = g2r - bc[1:2] - corrm * (dn00 * dn00)
        cx_ref[pl.ds(r, 1), :] = mxr + bc[2:3]
        return acc

    jax.lax.fori_loop(0, _R, loop3, jnp.int32(0))


def _edge_kernel(gr_ref, gc_ref, out_ref):
    gr = gr_ref[...]
    gc = gc_ref[...]
    dt = gr[0:1] - gc[0:1]
    dv = gr[1:4] - gc[1:4]
    d = jnp.sqrt(jnp.sum(dv * dv, axis=0, keepdims=True))
    corr = jnp.sum(gr[4:7] * gc[4:7], axis=0, keepdims=True)
    dcd = gr[7:8] - gc[7:8]
    dcm = gr[8:9] - gc[8:9]
    out_ref[...] = jnp.concatenate(
        [dt, d, corr, jnp.sign(dcd), jnp.abs(dcd),
         jnp.sign(dcm), jnp.abs(dcm), jnp.zeros_like(dt)], axis=0)


def kernel(P, B, row, col):
    f32 = jnp.float32
    Bi = B.astype(jnp.int32)
    B2 = Bi.reshape(_R, _C)
    Px = P[:, 0].reshape(_R, _C).astype(f32)
    Py = P[:, 1].reshape(_R, _C).astype(f32)
    Pz = P[:, 2].reshape(_R, _C).astype(f32)

    shp = jax.ShapeDtypeStruct((_R, _C), f32)
    outs = pl.pallas_call(
        _node_kernel,
        out_shape=(shp, shp, shp, shp, shp, shp, shp, shp, shp, shp, shp,
                   jax.ShapeDtypeStruct((8, _S), f32)),
        scratch_shapes=[pltpu.VMEM((_R, _C), f32)] * 4,
    )(B2, Px, Py, Pz)
    tn, cd, cm, dn, cx, px, py, pz, dx, dy, dz, seg = outs

    X = jnp.stack([tn, cd, cm, dn, cx], axis=0).reshape(5, _R * _C).T

    tableT = jnp.concatenate(
        [tn.reshape(1, -1), px.reshape(1, -1), py.reshape(1, -1),
         pz.reshape(1, -1), dx.reshape(1, -1), dy.reshape(1, -1),
         dz.reshape(1, -1), cd.reshape(1, -1), cm.reshape(1, -1),
         jnp.zeros((7, _R * _C), f32)], axis=0)
    Gr = jnp.take(tableT, row.astype(jnp.int32), axis=1)
    Gc = jnp.take(tableT, col.astype(jnp.int32), axis=1)

    E8 = pl.pallas_call(
        _edge_kernel,
        grid=(_NE // _EB,),
        in_specs=[pl.BlockSpec((16, _EB), lambda i: (0, i)),
                  pl.BlockSpec((16, _EB), lambda i: (0, i))],
        out_specs=pl.BlockSpec((8, _EB), lambda i: (0, i)),
        out_shape=jax.ShapeDtypeStruct((8, _NE), f32),
    )(Gr, Gc)
    E = E8[:7].T

    P_STD = seg[0]
    log_L = seg[1]
    u = seg[2:5].T
    return (X, E, P_STD, log_L, u)
